# baseline (speedup 1.0000x reference)
import functools

import jax
import jax.numpy as jnp
from jax import lax
from jax.experimental import pallas as pl
from jax.experimental.pallas import tpu as pltpu
from jax.experimental.pallas import tpu_sc as plsc

NC = 2
NS = 16
NW = NC * NS
K = 16
BR = 8


@functools.cache
def _make_gather_kernel(S, V, D, BCH):
    BB = BCH // NW
    tpw = S * BB
    mesh = plsc.VectorSubcoreMesh(
        core_axis_name="c", subcore_axis_name="s", num_cores=NC, num_subcores=NS
    )

    @functools.partial(
        pl.kernel,
        out_type=jax.ShapeDtypeStruct((S, BCH, D), jnp.float32),
        mesh=mesh,
        scratch_types=[
            pltpu.VMEM((tpw,), jnp.int32),
            [pltpu.VMEM((BB, D), jnp.float32)] * 2,
            [pltpu.SemaphoreType.DMA] * 2,
            [pltpu.SemaphoreType.DMA] * 2,
        ],
    )
    def gk(xu_ref, tok_ref, emb_ref, idx_v, bufs, gsems, osems):
        wid = lax.axis_index("s") * NC + lax.axis_index("c")
        col = pl.multiple_of(wid * BB, BB)

        pltpu.sync_copy(xu_ref.at[pl.ds(wid * tpw, tpw)], idx_v)

        def issue_gather(s, b):
            pltpu.async_copy(
                tok_ref.at[idx_v.at[pl.ds(s * BB, BB)]], bufs[b], gsems[b])

        issue_gather(0, 0)

        def s_step(s, b):
            pltpu.make_async_copy(
                tok_ref.at[idx_v.at[pl.ds(s * BB, BB)]],
                bufs[b], gsems[b]).wait()
            pltpu.async_copy(bufs[b], emb_ref.at[s, pl.ds(col, BB)], osems[b])

            @pl.when(s >= 1)
            def _():
                pltpu.make_async_copy(
                    bufs[1 - b], emb_ref.at[0, pl.ds(0, BB)],
                    osems[1 - b]).wait()

            @pl.when(s + 1 < S)
            def _():
                issue_gather(s + 1, 1 - b)

        def outer(o, _):
            for b in range(2):
                s_step(o * 2 + b, b)
            return 0

        lax.fori_loop(0, S // 2, outer, 0)
        b_last = (S - 1) % 2
        pltpu.make_async_copy(
            bufs[b_last], emb_ref.at[0, pl.ds(0, BB)], osems[b_last]).wait()

    return gk


def _ln_body(emb_ref, pos_ref, g_ref, b_ref, *rest):
    out_ref = rest[-1]
    S = emb_ref.shape[0]
    e = emb_ref[...] + pos_ref[0:S, :][:, None, :]
    mean = jnp.mean(e, axis=2, keepdims=True)
    c = e - mean
    var = jnp.mean(c * c, axis=2, keepdims=True)
    out_ref[...] = (c * lax.rsqrt(var + 1e-5) * g_ref[0][None, None, :]
                    + b_ref[0][None, None, :])


def _make_ln_call(k_idx, S, B, D, BCH, SP, aliased):
    nblk = BCH // BR
    base = k_idx * nblk
    out_spec = pl.BlockSpec((S, BR, D), lambda g: (0, base + g, 0))
    in_specs = [
        pl.BlockSpec((S, BR, D), lambda g: (0, g, 0)),
        pl.BlockSpec((SP, D), lambda g: (0, 0)),
        pl.BlockSpec((1, D), lambda g: (0, 0)),
        pl.BlockSpec((1, D), lambda g: (0, 0)),
    ]
    kwargs = {}
    if aliased:
        in_specs.append(pl.BlockSpec(memory_space=pl.ANY))
        kwargs["input_output_aliases"] = {4: 0}
    return pl.pallas_call(
        _ln_body,
        grid=(nblk,),
        in_specs=in_specs,
        out_specs=out_spec,
        out_shape=jax.ShapeDtypeStruct((S, B, D), jnp.float32),
        **kwargs,
    )


def kernel(x, tok_table, pos_table, gamma, beta):
    B, S = x.shape
    V, D = tok_table.shape
    SP = pos_table.shape[0]
    BCH = B // K
    BB = BCH // NW
    g2 = gamma.reshape(1, D)
    b2 = beta.reshape(1, D)
    gk = _make_gather_kernel(S, V, D, BCH)
    out = None
    for k in range(K):
        xu = (x[k * BCH:(k + 1) * BCH].reshape(NW, BB, S)
              .transpose(0, 2, 1).reshape(-1))
        emb = gk(xu, tok_table)
        ln = _make_ln_call(k, S, B, D, BCH, SP, aliased=k > 0)
        args = (emb, pos_table, g2, b2) + ((out,) if k > 0 else ())
        out = ln(*args)
    return out.transpose(1, 0, 2)


# speedup vs baseline: 1.2570x; 1.2570x over previous
import functools

import jax
import jax.numpy as jnp
from jax import lax
from jax.experimental import pallas as pl
from jax.experimental.pallas import tpu as pltpu
from jax.experimental.pallas import tpu_sc as plsc

NC = 2
NS = 16
NW = NC * NS
K = 8
BR = 8


@functools.cache
def _make_gather_kernel(S, V, D, BCH):
    BB = BCH // NW
    tpw = S * BB
    mesh = plsc.VectorSubcoreMesh(
        core_axis_name="c", subcore_axis_name="s", num_cores=NC, num_subcores=NS
    )

    @functools.partial(
        pl.kernel,
        out_type=jax.ShapeDtypeStruct((S, BCH, D), jnp.float32),
        mesh=mesh,
        scratch_types=[
            pltpu.VMEM((tpw,), jnp.int32),
            [pltpu.VMEM((BB, D), jnp.float32)] * 2,
            [pltpu.SemaphoreType.DMA] * 2,
            [pltpu.SemaphoreType.DMA] * 2,
        ],
    )
    def gk(xu_ref, tok_ref, emb_ref, idx_v, bufs, gsems, osems):
        wid = lax.axis_index("s") * NC + lax.axis_index("c")
        col = pl.multiple_of(wid * BB, BB)

        pltpu.sync_copy(xu_ref.at[pl.ds(wid * tpw, tpw)], idx_v)

        def issue_gather(s, b):
            pltpu.async_copy(
                tok_ref.at[idx_v.at[pl.ds(s * BB, BB)]], bufs[b], gsems[b])

        issue_gather(0, 0)

        def s_step(s, b):
            pltpu.make_async_copy(
                tok_ref.at[idx_v.at[pl.ds(s * BB, BB)]],
                bufs[b], gsems[b]).wait()
            pltpu.async_copy(bufs[b], emb_ref.at[s, pl.ds(col, BB)], osems[b])

            @pl.when(s >= 1)
            def _():
                pltpu.make_async_copy(
                    bufs[1 - b], emb_ref.at[0, pl.ds(0, BB)],
                    osems[1 - b]).wait()

            @pl.when(s + 1 < S)
            def _():
                issue_gather(s + 1, 1 - b)

        def outer(o, _):
            for b in range(2):
                s_step(o * 2 + b, b)
            return 0

        lax.fori_loop(0, S // 2, outer, 0)
        b_last = (S - 1) % 2
        pltpu.make_async_copy(
            bufs[b_last], emb_ref.at[0, pl.ds(0, BB)], osems[b_last]).wait()

    return gk


def _ln_body(emb_ref, pos_ref, g_ref, b_ref, *rest):
    out_ref = rest[-1]
    S = emb_ref.shape[0]
    e = emb_ref[...] + pos_ref[0:S, :][:, None, :]
    mean = jnp.mean(e, axis=2, keepdims=True)
    c = e - mean
    var = jnp.mean(c * c, axis=2, keepdims=True)
    out_ref[...] = (c * lax.rsqrt(var + 1e-5) * g_ref[0][None, None, :]
                    + b_ref[0][None, None, :])


def _make_ln_call(k_idx, S, B, D, BCH, SP, aliased):
    nblk = BCH // BR
    base = k_idx * nblk
    out_spec = pl.BlockSpec((S, BR, D), lambda g: (0, base + g, 0))
    in_specs = [
        pl.BlockSpec((S, BR, D), lambda g: (0, g, 0)),
        pl.BlockSpec((SP, D), lambda g: (0, 0)),
        pl.BlockSpec((1, D), lambda g: (0, 0)),
        pl.BlockSpec((1, D), lambda g: (0, 0)),
    ]
    kwargs = {}
    if aliased:
        in_specs.append(pl.BlockSpec(memory_space=pl.ANY))
        kwargs["input_output_aliases"] = {4: 0}
    return pl.pallas_call(
        _ln_body,
        grid=(nblk,),
        in_specs=in_specs,
        out_specs=out_spec,
        out_shape=jax.ShapeDtypeStruct((S, B, D), jnp.float32),
        **kwargs,
    )


def kernel(x, tok_table, pos_table, gamma, beta):
    B, S = x.shape
    V, D = tok_table.shape
    SP = pos_table.shape[0]
    BCH = B // K
    BB = BCH // NW
    g2 = gamma.reshape(1, D)
    b2 = beta.reshape(1, D)
    gk = _make_gather_kernel(S, V, D, BCH)
    out = None
    for k in range(K):
        xu = (x[k * BCH:(k + 1) * BCH].reshape(NW, BB, S)
              .transpose(0, 2, 1).reshape(-1))
        emb = gk(xu, tok_table)
        ln = _make_ln_call(k, S, B, D, BCH, SP, aliased=k > 0)
        args = (emb, pos_table, g2, b2) + ((out,) if k > 0 else ())
        out = ln(*args)
    return out.transpose(1, 0, 2)
